# baseline (speedup 1.0000x reference)
import functools
import math

import jax
import jax.numpy as jnp
from jax import lax
from jax.experimental import pallas as pl
from jax.experimental.pallas import tpu as pltpu
from jax.experimental.pallas import tpu_sc as plsc

TOPK = 32
RPAD = 64
NQ = 8192
NS = 8192
CHUNK = 128
NCHUNK = NS // CHUNK
NUM_WORKERS = 32
NGROUPS = 8


def _proj_body(x_ref, wt_ref, o_ref):
    o_ref[...] = jnp.dot(x_ref[...], wt_ref[...],
                         preferred_element_type=jnp.float32)


def _project(x, wt, br=1024):
    n = x.shape[0]
    d = x.shape[1]
    return pl.pallas_call(
        _proj_body,
        grid=(n // br,),
        in_specs=[pl.BlockSpec((br, d), lambda i: (i, 0)),
                  pl.BlockSpec((d, RPAD), lambda i: (0, 0))],
        out_specs=pl.BlockSpec((br, RPAD), lambda i: (i, 0)),
        out_shape=jax.ShapeDtypeStruct((n, RPAD), jnp.float32),
    )(x, wt)


def _score_body(scale, q_ref, wt_ref, rkt_ref, mask_ref, s_ref, cm_ref):
    rq = jnp.dot(q_ref[...], wt_ref[...], preferred_element_type=jnp.float32)
    s = jnp.dot(rq, rkt_ref[...], preferred_element_type=jnp.float32)
    s = s * scale + mask_ref[...]
    s_ref[...] = s
    br = s.shape[0]
    cm_ref[...] = jnp.max(s.reshape(br, NCHUNK, CHUNK), axis=2)


def _scores(q2, wt, rkt, mask2d, scale, br=256):
    nq = q2.shape[0]
    d = q2.shape[1]
    grid = nq // br
    return pl.pallas_call(
        functools.partial(_score_body, scale),
        grid=(grid,),
        in_specs=[pl.BlockSpec((br, d), lambda i: (i, 0)),
                  pl.BlockSpec((d, RPAD), lambda i: (0, 0)),
                  pl.BlockSpec((RPAD, NS), lambda i: (0, 0)),
                  pl.BlockSpec((1, NS), lambda i: (0, 0))],
        out_specs=[pl.BlockSpec((br, NS), lambda i: (i, 0)),
                   pl.BlockSpec((br, NCHUNK), lambda i: (i, 0))],
        out_shape=[jax.ShapeDtypeStruct((nq, NS), jnp.float32),
                   jax.ShapeDtypeStruct((nq, NCHUNK), jnp.float32)],
    )(q2, wt, rkt, mask2d)


def _topk_body(rpw, scores_hbm, cmax_hbm, idx_hbm, val_hbm,
               row_a, row_b, row_c, row_d, m_all, idx_acc, val_acc,
               sem_a, sem_b, sem_c, sem_d):
    cc = lax.axis_index("c")
    ss = lax.axis_index("s")
    wid = ss * 2 + cc
    base = wid * rpw
    iota = lax.broadcasted_iota(jnp.int32, (16,), 0)
    lane0 = iota == 0
    NEG = jnp.float32(-jnp.inf)
    BIG = jnp.int32(1 << 30)
    NEG_VEC = jnp.full((16,), NEG, jnp.float32)

    def _putv(ref, r, pos_v, val_v):
        plsc.store_scatter(ref, [jnp.full((16,), r, jnp.int32), pos_v],
                           val_v, mask=lane0)

    def _shuf(x, s):
        return x.at[iota ^ s].get(mode="promise_in_bounds")

    def _lanemax(x):
        for sh in (8, 4, 2, 1):
            x = jnp.maximum(x, _shuf(x, sh))
        return x

    pltpu.sync_copy(cmax_hbm.at[pl.ds(base, rpw)], m_all)
    pltpu.async_copy(scores_hbm.at[base], row_a, sem_a)
    pltpu.async_copy(scores_hbm.at[base + 1], row_b, sem_b)

    def step(i, m, r, row_v):
        mmv = jnp.maximum(jnp.maximum(m[0], m[1]),
                          jnp.maximum(m[2], m[3]))
        cmax_v = _lanemax(mmv)
        cand = None
        for j in range(4):
            fj = plsc.all_reduce_ffs(m[j] == cmax_v)
            cj = jnp.where(fj < 16, fj + (16 * j), BIG)
            cand = cj if cand is None else jnp.minimum(cand, cj)
        cid_v = cand
        start = cid_v[0] * CHUNK
        xs, pos = [], None
        for j in range(8):
            x = row_v[pl.ds(start + 16 * j, 16)]
            xs.append(x)
            fj = plsc.all_reduce_ffs(x == cmax_v)
            pj = jnp.where(fj < 16, (start + 16 * j) + fj, BIG)
            pos = pj if pos is None else jnp.minimum(pos, pj)
        p_v = pos
        nm = None
        for j in range(8):
            d = p_v - (start + 16 * j)
            xm = jnp.where(iota == d, NEG, xs[j])
            nm = xm if nm is None else jnp.maximum(nm, xm)
        newmax = jnp.max(nm)
        plsc.store_scatter(row_v, [p_v], NEG_VEC, mask=lane0)
        ivec = jnp.full((16,), i, jnp.int32)
        _putv(idx_acc, r, ivec, p_v)
        _putv(val_acc, r, ivec, cmax_v)
        cdiv = cid_v >> 4
        cmod = cid_v & 15
        return tuple(
            jnp.where((iota == cmod) & (cdiv == j), newmax, m[j])
            for j in range(4))

    def process_pair(r, row_x, row_y):
        def it_body(i, m):
            ma = step(i, m[:4], r, row_x)
            mb = step(i, m[4:], r + 1, row_y)
            return ma + mb

        m0 = tuple(m_all[r, pl.ds(16 * j, 16)] for j in range(4))
        m1 = tuple(m_all[r + 1, pl.ds(16 * j, 16)] for j in range(4))
        lax.fori_loop(0, TOPK, it_body, m0 + m1)

    def body4(q, carry):
        r0 = 4 * q
        pltpu.async_copy(scores_hbm.at[base + r0 + 2], row_c, sem_c)
        pltpu.async_copy(scores_hbm.at[base + r0 + 3], row_d, sem_d)
        pltpu.make_async_copy(scores_hbm.at[base + r0], row_a, sem_a).wait()
        pltpu.make_async_copy(scores_hbm.at[base + r0 + 1], row_b, sem_b).wait()
        process_pair(r0, row_a, row_b)

        @pl.when(q < rpw // 4 - 1)
        def _():
            pltpu.async_copy(scores_hbm.at[base + r0 + 4], row_a, sem_a)
            pltpu.async_copy(scores_hbm.at[base + r0 + 5], row_b, sem_b)

        pltpu.make_async_copy(scores_hbm.at[base + r0 + 2], row_c, sem_c).wait()
        pltpu.make_async_copy(scores_hbm.at[base + r0 + 3], row_d, sem_d).wait()
        process_pair(r0 + 2, row_c, row_d)
        return carry

    lax.fori_loop(0, rpw // 4, body4, 0)
    pltpu.sync_copy(idx_acc, idx_hbm.at[pl.ds(base, rpw)])
    pltpu.sync_copy(val_acc, val_hbm.at[pl.ds(base, rpw)])


def _topk(scores, cmax):
    nq = scores.shape[0]
    rpw = nq // NUM_WORKERS
    mesh = plsc.VectorSubcoreMesh(core_axis_name="c", subcore_axis_name="s")
    fn = pl.kernel(
        functools.partial(_topk_body, rpw),
        out_type=[jax.ShapeDtypeStruct((nq, TOPK), jnp.int32),
                  jax.ShapeDtypeStruct((nq, TOPK), jnp.float32)],
        mesh=mesh,
        compiler_params=pltpu.CompilerParams(needs_layout_passes=False),
        scratch_types=[pltpu.VMEM((NS,), jnp.float32),
                       pltpu.VMEM((NS,), jnp.float32),
                       pltpu.VMEM((NS,), jnp.float32),
                       pltpu.VMEM((NS,), jnp.float32),
                       pltpu.VMEM((rpw, NCHUNK), jnp.float32),
                       pltpu.VMEM((rpw, TOPK), jnp.int32),
                       pltpu.VMEM((rpw, TOPK), jnp.float32),
                       pltpu.SemaphoreType.DMA,
                       pltpu.SemaphoreType.DMA,
                       pltpu.SemaphoreType.DMA,
                       pltpu.SemaphoreType.DMA],
    )
    return fn(scores, cmax)


def kernel(query, slot_keys, reliability_mask, W_router):
    b, s, d = query.shape
    r = W_router.shape[0]
    scale = 1.0 / math.sqrt(r)
    q2 = query.reshape(b * s, d)
    wt = jnp.zeros((d, RPAD), jnp.float32).at[:, :r].set(W_router.T)
    rk = _project(slot_keys, wt)
    rkt = rk.T
    mask2d = reliability_mask.reshape(1, NS)
    gsz = (b * s) // NGROUPS
    outs = []
    for g in range(NGROUPS):
        sc_g, cm_g = _scores(q2[g * gsz:(g + 1) * gsz], wt, rkt, mask2d,
                             scale)
        outs.append(_topk(sc_g, cm_g))
    idx = jnp.concatenate([o[0] for o in outs])
    val = jnp.concatenate([o[1] for o in outs])
    return idx.reshape(b, s, TOPK), val.reshape(b, s, TOPK)


# speedup vs baseline: 1.0588x; 1.0588x over previous
import functools
import math

import jax
import jax.numpy as jnp
from jax import lax
from jax.experimental import pallas as pl
from jax.experimental.pallas import tpu as pltpu
from jax.experimental.pallas import tpu_sc as plsc

TOPK = 32
RPAD = 64
NQ = 8192
NS = 8192
CHUNK = 128
NCHUNK = NS // CHUNK
NUM_WORKERS = 32
NGROUPS = 4


def _proj_body(x_ref, wt_ref, o_ref):
    o_ref[...] = jnp.dot(x_ref[...], wt_ref[...],
                         preferred_element_type=jnp.float32)


def _project(x, wt, br=1024):
    n = x.shape[0]
    d = x.shape[1]
    return pl.pallas_call(
        _proj_body,
        grid=(n // br,),
        in_specs=[pl.BlockSpec((br, d), lambda i: (i, 0)),
                  pl.BlockSpec((d, RPAD), lambda i: (0, 0))],
        out_specs=pl.BlockSpec((br, RPAD), lambda i: (i, 0)),
        out_shape=jax.ShapeDtypeStruct((n, RPAD), jnp.float32),
    )(x, wt)


def _score_body(scale, q_ref, wt_ref, rkt_ref, mask_ref, s_ref, cm_ref):
    rq = jnp.dot(q_ref[...], wt_ref[...], preferred_element_type=jnp.float32)
    s = jnp.dot(rq, rkt_ref[...], preferred_element_type=jnp.float32)
    s = s * scale + mask_ref[...]
    s_ref[...] = s
    br = s.shape[0]
    cm_ref[...] = jnp.max(s.reshape(br, NCHUNK, CHUNK), axis=2)


def _scores(q2, wt, rkt, mask2d, scale, br=256):
    nq = q2.shape[0]
    d = q2.shape[1]
    grid = nq // br
    return pl.pallas_call(
        functools.partial(_score_body, scale),
        grid=(grid,),
        in_specs=[pl.BlockSpec((br, d), lambda i: (i, 0)),
                  pl.BlockSpec((d, RPAD), lambda i: (0, 0)),
                  pl.BlockSpec((RPAD, NS), lambda i: (0, 0)),
                  pl.BlockSpec((1, NS), lambda i: (0, 0))],
        out_specs=[pl.BlockSpec((br, NS), lambda i: (i, 0)),
                   pl.BlockSpec((br, NCHUNK), lambda i: (i, 0))],
        out_shape=[jax.ShapeDtypeStruct((nq, NS), jnp.float32),
                   jax.ShapeDtypeStruct((nq, NCHUNK), jnp.float32)],
    )(q2, wt, rkt, mask2d)


def _topk_body(rpw, scores_hbm, cmax_hbm, idx_hbm, val_hbm,
               row_a, row_b, row_c, row_d, row_e, row_f, row_g, row_h,
               m_all, idx_acc, val_acc,
               sem_a, sem_b, sem_c, sem_d, sem_e, sem_f, sem_g, sem_h):
    cc = lax.axis_index("c")
    ss = lax.axis_index("s")
    wid = ss * 2 + cc
    base = wid * rpw
    iota = lax.broadcasted_iota(jnp.int32, (16,), 0)
    lane0 = iota == 0
    NEG = jnp.float32(-jnp.inf)
    BIG = jnp.int32(1 << 30)
    NEG_VEC = jnp.full((16,), NEG, jnp.float32)

    def _putv(ref, r, pos_v, val_v):
        plsc.store_scatter(ref, [jnp.full((16,), r, jnp.int32), pos_v],
                           val_v, mask=lane0)

    def _shuf(x, s):
        return x.at[iota ^ s].get(mode="promise_in_bounds")

    def _lanemax(x):
        for sh in (8, 4, 2, 1):
            x = jnp.maximum(x, _shuf(x, sh))
        return x

    pltpu.sync_copy(cmax_hbm.at[pl.ds(base, rpw)], m_all)
    pltpu.async_copy(scores_hbm.at[base], row_a, sem_a)
    pltpu.async_copy(scores_hbm.at[base + 1], row_b, sem_b)
    pltpu.async_copy(scores_hbm.at[base + 2], row_c, sem_c)
    pltpu.async_copy(scores_hbm.at[base + 3], row_d, sem_d)

    def step(i, m, r, row_v):
        mmv = jnp.maximum(jnp.maximum(m[0], m[1]),
                          jnp.maximum(m[2], m[3]))
        cmax_v = _lanemax(mmv)
        cand = None
        for j in range(4):
            fj = plsc.all_reduce_ffs(m[j] == cmax_v)
            cj = jnp.where(fj < 16, fj + (16 * j), BIG)
            cand = cj if cand is None else jnp.minimum(cand, cj)
        cid_v = cand
        start = cid_v[0] * CHUNK
        pos = None
        for j in range(8):
            x = row_v[pl.ds(start + 16 * j, 16)]
            fj = plsc.all_reduce_ffs(x == cmax_v)
            pj = jnp.where(fj < 16, (start + 16 * j) + fj, BIG)
            pos = pj if pos is None else jnp.minimum(pos, pj)
        p_v = pos
        nm = None
        for j in range(8):
            x = row_v[pl.ds(start + 16 * j, 16)]
            d = p_v - (start + 16 * j)
            xm = jnp.where(iota == d, NEG, x)
            nm = xm if nm is None else jnp.maximum(nm, xm)
        newmax = jnp.max(nm)
        plsc.store_scatter(row_v, [p_v], NEG_VEC, mask=lane0)
        ivec = jnp.full((16,), i, jnp.int32)
        _putv(idx_acc, r, ivec, p_v)
        _putv(val_acc, r, ivec, cmax_v)
        cdiv = cid_v >> 4
        cmod = cid_v & 15
        return tuple(
            jnp.where((iota == cmod) & (cdiv == j), newmax, m[j])
            for j in range(4))

    def process_quad(r, rows):
        def it_body(i, m):
            out = ()
            for k in range(4):
                out += step(i, m[4 * k:4 * k + 4], r + k, rows[k])
            return out

        m0 = ()
        for k in range(4):
            m0 += tuple(m_all[r + k, pl.ds(16 * j, 16)] for j in range(4))
        lax.fori_loop(0, TOPK, it_body, m0)

    bufs = (row_a, row_b, row_c, row_d)
    sems = (sem_a, sem_b, sem_c, sem_d)
    bufs2 = (row_e, row_f, row_g, row_h)
    sems2 = (sem_e, sem_f, sem_g, sem_h)

    def body8(q, carry):
        r0 = 8 * q
        for k in range(4):
            pltpu.async_copy(scores_hbm.at[base + r0 + 4 + k],
                             bufs2[k], sems2[k])
        for k in range(4):
            pltpu.make_async_copy(scores_hbm.at[base + r0 + k],
                                  bufs[k], sems[k]).wait()
        process_quad(r0, bufs)

        @pl.when(q < rpw // 8 - 1)
        def _():
            for k in range(4):
                pltpu.async_copy(scores_hbm.at[base + r0 + 8 + k],
                                 bufs[k], sems[k])

        for k in range(4):
            pltpu.make_async_copy(scores_hbm.at[base + r0 + 4 + k],
                                  bufs2[k], sems2[k]).wait()
        process_quad(r0 + 4, bufs2)
        return carry

    lax.fori_loop(0, rpw // 8, body8, 0)
    pltpu.sync_copy(idx_acc, idx_hbm.at[pl.ds(base, rpw)])
    pltpu.sync_copy(val_acc, val_hbm.at[pl.ds(base, rpw)])


def _topk(scores, cmax):
    nq = scores.shape[0]
    rpw = nq // NUM_WORKERS
    mesh = plsc.VectorSubcoreMesh(core_axis_name="c", subcore_axis_name="s")
    fn = pl.kernel(
        functools.partial(_topk_body, rpw),
        out_type=[jax.ShapeDtypeStruct((nq, TOPK), jnp.int32),
                  jax.ShapeDtypeStruct((nq, TOPK), jnp.float32)],
        mesh=mesh,
        compiler_params=pltpu.CompilerParams(needs_layout_passes=False),
        scratch_types=[pltpu.VMEM((NS,), jnp.float32)] * 8 +
                      [pltpu.VMEM((rpw, NCHUNK), jnp.float32),
                       pltpu.VMEM((rpw, TOPK), jnp.int32),
                       pltpu.VMEM((rpw, TOPK), jnp.float32)] +
                      [pltpu.SemaphoreType.DMA] * 8,
    )
    return fn(scores, cmax)


def kernel(query, slot_keys, reliability_mask, W_router):
    b, s, d = query.shape
    r = W_router.shape[0]
    scale = 1.0 / math.sqrt(r)
    q2 = query.reshape(b * s, d)
    wt = jnp.zeros((d, RPAD), jnp.float32).at[:, :r].set(W_router.T)
    rk = _project(slot_keys, wt)
    rkt = rk.T
    mask2d = reliability_mask.reshape(1, NS)
    gsz = (b * s) // NGROUPS
    outs = []
    for g in range(NGROUPS):
        sc_g, cm_g = _scores(q2[g * gsz:(g + 1) * gsz], wt, rkt, mask2d,
                             scale)
        outs.append(_topk(sc_g, cm_g))
    idx = jnp.concatenate([o[0] for o in outs])
    val = jnp.concatenate([o[1] for o in outs])
    return idx.reshape(b, s, TOPK), val.reshape(b, s, TOPK)


# speedup vs baseline: 1.0627x; 1.0038x over previous
import functools
import math

import jax
import jax.numpy as jnp
from jax import lax
from jax.experimental import pallas as pl
from jax.experimental.pallas import tpu as pltpu
from jax.experimental.pallas import tpu_sc as plsc

TOPK = 32
RPAD = 64
NQ = 8192
NS = 8192
CHUNK = 128
NCHUNK = NS // CHUNK
NUM_WORKERS = 32
NGROUPS = 4


def _proj_body(x_ref, wt_ref, o_ref):
    o_ref[...] = jnp.dot(x_ref[...], wt_ref[...],
                         preferred_element_type=jnp.float32)


def _project(x, wt, br=1024):
    n = x.shape[0]
    d = x.shape[1]
    return pl.pallas_call(
        _proj_body,
        grid=(n // br,),
        in_specs=[pl.BlockSpec((br, d), lambda i: (i, 0)),
                  pl.BlockSpec((d, RPAD), lambda i: (0, 0))],
        out_specs=pl.BlockSpec((br, RPAD), lambda i: (i, 0)),
        out_shape=jax.ShapeDtypeStruct((n, RPAD), jnp.float32),
    )(x, wt)


def _score_body(scale, q_ref, wt_ref, rkt_ref, mask_ref, s_ref, cm_ref):
    rq = jnp.dot(q_ref[...], wt_ref[...], preferred_element_type=jnp.float32)
    s = jnp.dot(rq, rkt_ref[...], preferred_element_type=jnp.float32)
    s = s * scale + mask_ref[...]
    s_ref[...] = s
    br = s.shape[0]
    cm_ref[...] = jnp.max(s.reshape(br, NCHUNK, CHUNK), axis=2)


def _scores(q2, wt, rkt, mask2d, scale, br=512):
    nq = q2.shape[0]
    d = q2.shape[1]
    grid = nq // br
    return pl.pallas_call(
        functools.partial(_score_body, scale),
        grid=(grid,),
        in_specs=[pl.BlockSpec((br, d), lambda i: (i, 0)),
                  pl.BlockSpec((d, RPAD), lambda i: (0, 0)),
                  pl.BlockSpec((RPAD, NS), lambda i: (0, 0)),
                  pl.BlockSpec((1, NS), lambda i: (0, 0))],
        out_specs=[pl.BlockSpec((br, NS), lambda i: (i, 0)),
                   pl.BlockSpec((br, NCHUNK), lambda i: (i, 0))],
        out_shape=[jax.ShapeDtypeStruct((nq, NS), jnp.float32),
                   jax.ShapeDtypeStruct((nq, NCHUNK), jnp.float32)],
    )(q2, wt, rkt, mask2d)


def _topk_body(rpw, scores_hbm, cmax_hbm, idx_hbm, val_hbm,
               row_a, row_b, row_c, row_d, row_e, row_f, row_g, row_h,
               m_all, idx_acc, val_acc,
               sem_a, sem_b, sem_c, sem_d, sem_e, sem_f, sem_g, sem_h):
    cc = lax.axis_index("c")
    ss = lax.axis_index("s")
    wid = ss * 2 + cc
    base = wid * rpw
    iota = lax.broadcasted_iota(jnp.int32, (16,), 0)
    lane0 = iota == 0
    NEG = jnp.float32(-jnp.inf)
    BIG = jnp.int32(1 << 30)
    NEG_VEC = jnp.full((16,), NEG, jnp.float32)

    def _putv(ref, r, pos_v, val_v):
        plsc.store_scatter(ref, [jnp.full((16,), r, jnp.int32), pos_v],
                           val_v, mask=lane0)

    def _shuf(x, s):
        return x.at[iota ^ s].get(mode="promise_in_bounds")

    def _lanemax(x):
        for sh in (8, 4, 2, 1):
            x = jnp.maximum(x, _shuf(x, sh))
        return x

    pltpu.sync_copy(cmax_hbm.at[pl.ds(base, rpw)], m_all)
    pltpu.async_copy(scores_hbm.at[base], row_a, sem_a)
    pltpu.async_copy(scores_hbm.at[base + 1], row_b, sem_b)
    pltpu.async_copy(scores_hbm.at[base + 2], row_c, sem_c)
    pltpu.async_copy(scores_hbm.at[base + 3], row_d, sem_d)

    def step(i, m, r, row_v):
        mmv = jnp.maximum(jnp.maximum(m[0], m[1]),
                          jnp.maximum(m[2], m[3]))
        cmax_v = _lanemax(mmv)
        cand = None
        for j in range(4):
            fj = plsc.all_reduce_ffs(m[j] == cmax_v)
            cj = jnp.where(fj < 16, fj + (16 * j), BIG)
            cand = cj if cand is None else jnp.minimum(cand, cj)
        cid_v = cand
        start = cid_v[0] * CHUNK
        pos = None
        for j in range(8):
            x = row_v[pl.ds(start + 16 * j, 16)]
            fj = plsc.all_reduce_ffs(x == cmax_v)
            pj = jnp.where(fj < 16, (start + 16 * j) + fj, BIG)
            pos = pj if pos is None else jnp.minimum(pos, pj)
        p_v = pos
        nm = None
        for j in range(8):
            x = row_v[pl.ds(start + 16 * j, 16)]
            d = p_v - (start + 16 * j)
            xm = jnp.where(iota == d, NEG, x)
            nm = xm if nm is None else jnp.maximum(nm, xm)
        newmax = jnp.max(nm)
        plsc.store_scatter(row_v, [p_v], NEG_VEC, mask=lane0)
        ivec = jnp.full((16,), i, jnp.int32)
        _putv(idx_acc, r, ivec, p_v)
        _putv(val_acc, r, ivec, cmax_v)
        cdiv = cid_v >> 4
        cmod = cid_v & 15
        return tuple(
            jnp.where((iota == cmod) & (cdiv == j), newmax, m[j])
            for j in range(4))

    def process_quad(r, rows):
        def it_body(i, m):
            out = ()
            for k in range(4):
                out += step(i, m[4 * k:4 * k + 4], r + k, rows[k])
            return out

        m0 = ()
        for k in range(4):
            m0 += tuple(m_all[r + k, pl.ds(16 * j, 16)] for j in range(4))
        lax.fori_loop(0, TOPK, it_body, m0)

    bufs = (row_a, row_b, row_c, row_d)
    sems = (sem_a, sem_b, sem_c, sem_d)
    bufs2 = (row_e, row_f, row_g, row_h)
    sems2 = (sem_e, sem_f, sem_g, sem_h)

    def body8(q, carry):
        r0 = 8 * q
        for k in range(4):
            pltpu.async_copy(scores_hbm.at[base + r0 + 4 + k],
                             bufs2[k], sems2[k])
        for k in range(4):
            pltpu.make_async_copy(scores_hbm.at[base + r0 + k],
                                  bufs[k], sems[k]).wait()
        process_quad(r0, bufs)

        @pl.when(q < rpw // 8 - 1)
        def _():
            for k in range(4):
                pltpu.async_copy(scores_hbm.at[base + r0 + 8 + k],
                                 bufs[k], sems[k])

        for k in range(4):
            pltpu.make_async_copy(scores_hbm.at[base + r0 + 4 + k],
                                  bufs2[k], sems2[k]).wait()
        process_quad(r0 + 4, bufs2)
        return carry

    lax.fori_loop(0, rpw // 8, body8, 0)
    pltpu.sync_copy(idx_acc, idx_hbm.at[pl.ds(base, rpw)])
    pltpu.sync_copy(val_acc, val_hbm.at[pl.ds(base, rpw)])


def _topk(scores, cmax):
    nq = scores.shape[0]
    rpw = nq // NUM_WORKERS
    mesh = plsc.VectorSubcoreMesh(core_axis_name="c", subcore_axis_name="s")
    fn = pl.kernel(
        functools.partial(_topk_body, rpw),
        out_type=[jax.ShapeDtypeStruct((nq, TOPK), jnp.int32),
                  jax.ShapeDtypeStruct((nq, TOPK), jnp.float32)],
        mesh=mesh,
        compiler_params=pltpu.CompilerParams(needs_layout_passes=False),
        scratch_types=[pltpu.VMEM((NS,), jnp.float32)] * 8 +
                      [pltpu.VMEM((rpw, NCHUNK), jnp.float32),
                       pltpu.VMEM((rpw, TOPK), jnp.int32),
                       pltpu.VMEM((rpw, TOPK), jnp.float32)] +
                      [pltpu.SemaphoreType.DMA] * 8,
    )
    return fn(scores, cmax)


def kernel(query, slot_keys, reliability_mask, W_router):
    b, s, d = query.shape
    r = W_router.shape[0]
    scale = 1.0 / math.sqrt(r)
    q2 = query.reshape(b * s, d)
    wt = jnp.zeros((d, RPAD), jnp.float32).at[:, :r].set(W_router.T)
    rk = _project(slot_keys, wt)
    rkt = rk.T
    mask2d = reliability_mask.reshape(1, NS)
    gsz = (b * s) // NGROUPS
    outs = []
    for g in range(NGROUPS):
        sc_g, cm_g = _scores(q2[g * gsz:(g + 1) * gsz], wt, rkt, mask2d,
                             scale)
        outs.append(_topk(sc_g, cm_g))
    idx = jnp.concatenate([o[0] for o in outs])
    val = jnp.concatenate([o[1] for o in outs])
    return idx.reshape(b, s, TOPK), val.reshape(b, s, TOPK)
